# baseline (speedup 1.0000x reference)
import functools

import jax
import jax.numpy as jnp
from jax import lax
from jax.experimental import pallas as pl
from jax.experimental.pallas import tpu as pltpu
from jax.experimental.pallas import tpu_sc as plsc

_B = 32
_K = 1024
_DIM = 50257
_KNN = 8
_NCLASS = 2
_INV_T = 20.0

_DB = 1024
_ND = -(-_DIM // _DB)


def _dist_body(q_ref, a_ref, out_ref, accc, acca):
    di = pl.program_id(0)
    nd = pl.num_programs(0)

    @pl.when(di == 0)
    def _init():
        accc[...] = jnp.zeros_like(accc)
        acca[...] = jnp.zeros_like(acca)

    dglob = di * _DB + lax.broadcasted_iota(jnp.int32, (1, _DB), 1)
    valid = dglob < _DIM
    a = jnp.where(valid, a_ref[...], 1.0)
    la = jnp.log(a)
    lq = jnp.log(jnp.where(valid, q_ref[...], 1.0))

    dn = (((1,), (1,)), ((), ()))
    accc[...] += lax.dot_general(lq, a, dn, preferred_element_type=jnp.float32)
    ones = jnp.ones((1, _DB), jnp.float32)
    acca[...] += lax.dot_general(ones, a * la, dn, preferred_element_type=jnp.float32)

    @pl.when(di == nd - 1)
    def _fin():
        out_ref[...] = (_INV_T / _DIM) * (accc[...] - acca[...])


def _scaled_dists(logits, queue_anchor):
    return pl.pallas_call(
        _dist_body,
        grid=(_ND,),
        in_specs=[
            pl.BlockSpec((_B, _DB), lambda di: (0, di)),
            pl.BlockSpec((_K, _DB), lambda di: (0, di)),
        ],
        out_specs=pl.BlockSpec((_B, _K), lambda di: (0, 0)),
        out_shape=jax.ShapeDtypeStruct((_B, _K), jnp.float32),
        scratch_shapes=[
            pltpu.VMEM((_B, _K), jnp.float32),
            pltpu.VMEM((1, _K), jnp.float32),
        ],
        compiler_params=pltpu.CompilerParams(
            dimension_semantics=("arbitrary",),
        ),
    )(logits, queue_anchor)


def _knn_tail(scaled, queue_label):
    info = plsc.get_sparse_core_info()
    nc, ns = info.num_cores, info.num_subcores
    assert nc * ns == _B

    mesh = plsc.VectorSubcoreMesh(core_axis_name="c", subcore_axis_name="s")

    @functools.partial(
        pl.kernel,
        mesh=mesh,
        out_type=jax.ShapeDtypeStruct((_B, 16), jnp.float32),
        scratch_types=[
            pltpu.VMEM((_K,), jnp.float32),
            pltpu.VMEM((_K,), jnp.int32),
            pltpu.VMEM((16,), jnp.float32),
        ],
        compiler_params=pltpu.CompilerParams(needs_layout_passes=False),
    )
    def tail(scaled_hbm, label_hbm, out_hbm, row_v, lab_v, out_v):
        b = lax.axis_index("s") * nc + lax.axis_index("c")
        pltpu.sync_copy(scaled_hbm.at[b], row_v)
        pltpu.sync_copy(label_hbm, lab_v)

        lane = lax.iota(jnp.int32, 16)
        rk = jnp.full((16,), -3.4e38, jnp.float32)
        rl = jnp.zeros((16,), jnp.int32)
        for c in range(_K // 16):
            ck = row_v[pl.ds(c * 16, 16)]
            cl = lab_v[pl.ds(c * 16, 16)]
            ck_s, cl_s = plsc.sort_key_val(ck, cl, descending=True)
            take_r = rk >= ck_s
            nk = jnp.where(take_r, rk, ck_s)
            nl = jnp.where(take_r, rl, cl_s)
            rk, rl = plsc.sort_key_val(nk, nl, descending=False)

        top8 = lane >= 8
        m = jnp.max(rk)
        w = jnp.where(top8, jnp.exp(rk - m), 0.0)
        s1 = jnp.sum(jnp.where(rl == 1, w, 0.0))
        s0 = jnp.sum(jnp.where(rl == 0, w, 0.0))
        d = jnp.full((16,), s0 + s1, jnp.float32)
        r = lax.bitcast_convert_type(
            jnp.full((16,), 0x7EF127EA, jnp.int32)
            - lax.bitcast_convert_type(d, jnp.int32), jnp.float32)
        for _ in range(3):
            r = r * (2.0 - d * r)
        out_v[...] = jnp.where(lane == 0, s0, jnp.where(lane == 1, s1, 0.0)) * r
        pltpu.sync_copy(out_v, out_hbm.at[b])

    return tail(scaled, queue_label)


def kernel(logits, queue_anchor, queue_label):
    scaled = _scaled_dists(logits, queue_anchor)
    out16 = _knn_tail(scaled, queue_label)
    return out16[:, :_NCLASS]


# speedup vs baseline: 1.0846x; 1.0846x over previous
import functools

import jax
import jax.numpy as jnp
from jax import lax
from jax.experimental import pallas as pl
from jax.experimental.pallas import tpu as pltpu
from jax.experimental.pallas import tpu_sc as plsc

_B = 32
_K = 1024
_DIM = 50257
_KNN = 8
_NCLASS = 2
_INV_T = 20.0

_KB = 128
_NK = _K // _KB


def _dist_body(q_ref, a_ref, out_ref, lq_s):
    ki = pl.program_id(0)

    @pl.when(ki == 0)
    def _init():
        lq_s[...] = jnp.log(q_ref[...])

    a = a_ref[...]
    la = jnp.log(a)

    dn = (((1,), (1,)), ((), ()))
    cross = lax.dot_general(lq_s[...], a, dn, preferred_element_type=jnp.float32)
    ones = jnp.ones((1, _DIM), jnp.float32)
    aloga = lax.dot_general(ones, a * la, dn, preferred_element_type=jnp.float32)
    out_ref[:, pl.ds(ki * _KB, _KB)] = (_INV_T / _DIM) * (cross - aloga)


def _scaled_dists(logits, queue_anchor):
    return pl.pallas_call(
        _dist_body,
        grid=(_NK,),
        in_specs=[
            pl.BlockSpec((_B, _DIM), lambda ki: (0, 0)),
            pl.BlockSpec((_KB, _DIM), lambda ki: (ki, 0)),
        ],
        out_specs=pl.BlockSpec((_B, _K), lambda ki: (0, 0)),
        out_shape=jax.ShapeDtypeStruct((_B, _K), jnp.float32),
        scratch_shapes=[
            pltpu.VMEM((_B, _DIM), jnp.float32),
        ],
        compiler_params=pltpu.CompilerParams(
            dimension_semantics=("arbitrary",),
            vmem_limit_bytes=120 * 1024 * 1024,
        ),
    )(logits, queue_anchor)


def _knn_tail(scaled, queue_label):
    info = plsc.get_sparse_core_info()
    nc, ns = info.num_cores, info.num_subcores
    assert nc * ns == _B

    mesh = plsc.VectorSubcoreMesh(core_axis_name="c", subcore_axis_name="s")

    @functools.partial(
        pl.kernel,
        mesh=mesh,
        out_type=jax.ShapeDtypeStruct((_B, 16), jnp.float32),
        scratch_types=[
            pltpu.VMEM((_K,), jnp.float32),
            pltpu.VMEM((_K,), jnp.int32),
            pltpu.VMEM((16,), jnp.float32),
        ],
        compiler_params=pltpu.CompilerParams(needs_layout_passes=False),
    )
    def tail(scaled_hbm, label_hbm, out_hbm, row_v, lab_v, out_v):
        b = lax.axis_index("s") * nc + lax.axis_index("c")
        pltpu.sync_copy(scaled_hbm.at[b], row_v)
        pltpu.sync_copy(label_hbm, lab_v)

        lane = lax.iota(jnp.int32, 16)
        rk = jnp.full((16,), -3.4e38, jnp.float32)
        rl = jnp.zeros((16,), jnp.int32)
        for c in range(_K // 16):
            ck = row_v[pl.ds(c * 16, 16)]
            cl = lab_v[pl.ds(c * 16, 16)]
            ck_s, cl_s = plsc.sort_key_val(ck, cl, descending=True)
            take_r = rk >= ck_s
            nk = jnp.where(take_r, rk, ck_s)
            nl = jnp.where(take_r, rl, cl_s)
            rk, rl = plsc.sort_key_val(nk, nl, descending=False)

        top8 = lane >= 8
        m = jnp.max(rk)
        w = jnp.where(top8, jnp.exp(rk - m), 0.0)
        s1 = jnp.sum(jnp.where(rl == 1, w, 0.0))
        s0 = jnp.sum(jnp.where(rl == 0, w, 0.0))
        d = jnp.full((16,), s0 + s1, jnp.float32)
        r = lax.bitcast_convert_type(
            jnp.full((16,), 0x7EF127EA, jnp.int32)
            - lax.bitcast_convert_type(d, jnp.int32), jnp.float32)
        for _ in range(3):
            r = r * (2.0 - d * r)
        out_v[...] = jnp.where(lane == 0, s0, jnp.where(lane == 1, s1, 0.0)) * r
        pltpu.sync_copy(out_v, out_hbm.at[b])

    return tail(scaled, queue_label)


def kernel(logits, queue_anchor, queue_label):
    scaled = _scaled_dists(logits, queue_anchor)
    out16 = _knn_tail(scaled, queue_label)
    return out16[:, :_NCLASS]
